# baseline (speedup 1.0000x reference)
import functools

import jax
import jax.numpy as jnp
from jax import lax
from jax.experimental import pallas as pl
from jax.experimental.pallas import tpu as pltpu
from jax.experimental.pallas import tpu_sc as plsc

EMBED_DIM = 64
POS_DIM = 32
OUT_DIM = EMBED_DIM + 2 * POS_DIM
B, L = 4096, 200
N_TOK = B * L

NC, NS = 2, 16
NW = NC * NS
TOK_PER_W = N_TOK // NW
T = 256
STEPS = TOK_PER_W // T
SHALF = STEPS // 2
SUB = 128
NSUB = T // SUB


def _emb_body(wid_hbm, p1_hbm, p2_hbm, wtab_hbm, ptab_hbm, out_hbm,
              widx, p1idx, p2idx, wbuf, p1buf, p2buf,
              isem, gsem, wsem):
    c = lax.axis_index("c")
    s = lax.axis_index("s")
    wid = s * NC + c
    base0 = wid * TOK_PER_W

    def idx_copies(i, b):
        row = pl.ds(base0 + i * T, T)
        return [
            pltpu.make_async_copy(wid_hbm.at[row], widx.at[b], isem.at[b]),
            pltpu.make_async_copy(p1_hbm.at[row], p1idx.at[b], isem.at[b]),
            pltpu.make_async_copy(p2_hbm.at[row], p2idx.at[b], isem.at[b]),
        ]

    def gather_copies(b):
        out = []
        for j in range(NSUB):
            sl = pl.ds(j * SUB, SUB)
            out.append(pltpu.make_async_copy(
                wtab_hbm.at[widx.at[b, sl]], wbuf.at[b, sl], gsem.at[b]))
            out.append(pltpu.make_async_copy(
                ptab_hbm.at[p1idx.at[b, sl]], p1buf.at[b, sl], gsem.at[b]))
            out.append(pltpu.make_async_copy(
                ptab_hbm.at[p2idx.at[b, sl]], p2buf.at[b, sl], gsem.at[b]))
        return out

    def write_copies(i, b):
        row = pl.ds(base0 + i * T, T)
        return [
            pltpu.make_async_copy(
                wbuf.at[b], out_hbm.at[row, pl.ds(0, EMBED_DIM)], wsem.at[b]),
            pltpu.make_async_copy(
                p1buf.at[b], out_hbm.at[row, pl.ds(EMBED_DIM, POS_DIM)],
                wsem.at[b]),
            pltpu.make_async_copy(
                p2buf.at[b], out_hbm.at[row, pl.ds(EMBED_DIM + POS_DIM, POS_DIM)],
                wsem.at[b]),
        ]

    def start(copies):
        for cp in copies:
            cp.start()

    def wait(copies):
        for cp in copies:
            cp.wait()

    start(idx_copies(0, 0))
    wait(idx_copies(0, 0))
    start(gather_copies(0))
    start(idx_copies(1, 1))
    wait(idx_copies(1, 1))
    start(gather_copies(1))
    wait(gather_copies(0))
    start(write_copies(0, 0))
    start(idx_copies(2, 0))

    def body(j, carry):
        i0 = 2 * j
        wait(idx_copies(i0, 0))
        wait(write_copies(i0 - 2, 0))
        start(gather_copies(0))
        wait(gather_copies(1))
        start(write_copies(i0 - 1, 1))
        start(idx_copies(i0 + 1, 1))
        wait(idx_copies(i0 + 1, 1))
        wait(write_copies(i0 - 1, 1))
        start(gather_copies(1))
        wait(gather_copies(0))
        start(write_copies(i0, 0))

        @pl.when(j < SHALF - 1)
        def _():
            start(idx_copies(i0 + 2, 0))

        return carry

    lax.fori_loop(1, SHALF, body, 0)

    wait(gather_copies(1))
    start(write_copies(STEPS - 1, 1))
    wait(write_copies(STEPS - 2, 0))
    wait(write_copies(STEPS - 1, 1))


@functools.partial(
    pl.kernel,
    out_type=jax.ShapeDtypeStruct((N_TOK, OUT_DIM), jnp.float32),
    mesh=plsc.VectorSubcoreMesh(core_axis_name="c", subcore_axis_name="s"),
    compiler_params=pltpu.CompilerParams(use_tc_tiling_on_sc=False),
    scratch_types=[
        pltpu.VMEM((2, T), jnp.int32),
        pltpu.VMEM((2, T), jnp.int32),
        pltpu.VMEM((2, T), jnp.int32),
        pltpu.VMEM((2, T, EMBED_DIM), jnp.float32),
        pltpu.VMEM((2, T, POS_DIM), jnp.float32),
        pltpu.VMEM((2, T, POS_DIM), jnp.float32),
        pltpu.SemaphoreType.DMA((2,)),
        pltpu.SemaphoreType.DMA((2,)),
        pltpu.SemaphoreType.DMA((2,)),
    ],
)
def _emb_kernel(*refs):
    _emb_body(*refs)


def kernel(word_id, pos_1, pos_2, word_table, pos_table):
    out = _emb_kernel(
        word_id.reshape(N_TOK),
        pos_1.reshape(N_TOK),
        pos_2.reshape(N_TOK),
        word_table,
        pos_table,
    )
    return out.reshape(B, L, OUT_DIM)


# speedup vs baseline: 1.0054x; 1.0054x over previous
import functools

import jax
import jax.numpy as jnp
from jax import lax
from jax.experimental import pallas as pl
from jax.experimental.pallas import tpu as pltpu
from jax.experimental.pallas import tpu_sc as plsc

EMBED_DIM = 64
POS_DIM = 32
POS_VOCAB = 201
OUT_DIM = EMBED_DIM + 2 * POS_DIM
B, L = 4096, 200

NC, NS = 2, 16
NW = NC * NS
ROWS_PER_W = B // NW
STEPS = ROWS_PER_W
SHALF = STEPS // 2
SPLITS = ((0, 128), (128, 72))


def _emb_body(wid_hbm, p1_hbm, p2_hbm, wtab_hbm, ptab_hbm, out_hbm,
              widx, p1idx, p2idx, wbuf, p1buf, p2buf,
              isem, gsem, wsem):
    c = lax.axis_index("c")
    s = lax.axis_index("s")
    wid = s * NC + c
    row0 = wid * ROWS_PER_W

    def idx_copies(i, b):
        bi = row0 + i
        return [
            pltpu.make_async_copy(wid_hbm.at[bi], widx.at[b], isem.at[b]),
            pltpu.make_async_copy(p1_hbm.at[bi], p1idx.at[b], isem.at[b]),
            pltpu.make_async_copy(p2_hbm.at[bi], p2idx.at[b], isem.at[b]),
        ]

    def gather_copies(b):
        out = []
        for off, n in SPLITS:
            sl = pl.ds(off, n)
            out.append(pltpu.make_async_copy(
                wtab_hbm.at[widx.at[b, sl]], wbuf.at[b, sl], gsem.at[b]))
            out.append(pltpu.make_async_copy(
                ptab_hbm.at[p1idx.at[b, sl]], p1buf.at[b, sl], gsem.at[b]))
            out.append(pltpu.make_async_copy(
                ptab_hbm.at[p2idx.at[b, sl]], p2buf.at[b, sl], gsem.at[b]))
        return out

    def write_copies(i, b):
        bi = row0 + i
        return [
            pltpu.make_async_copy(
                wbuf.at[b], out_hbm.at[bi, slice(None), pl.ds(0, EMBED_DIM)],
                wsem.at[b]),
            pltpu.make_async_copy(
                p1buf.at[b],
                out_hbm.at[bi, slice(None), pl.ds(EMBED_DIM, POS_DIM)],
                wsem.at[b]),
            pltpu.make_async_copy(
                p2buf.at[b],
                out_hbm.at[bi, slice(None), pl.ds(EMBED_DIM + POS_DIM, POS_DIM)],
                wsem.at[b]),
        ]

    def start(copies):
        for cp in copies:
            cp.start()

    def wait(copies):
        for cp in copies:
            cp.wait()

    start(idx_copies(0, 0))
    wait(idx_copies(0, 0))
    start(gather_copies(0))
    start(idx_copies(1, 1))
    wait(idx_copies(1, 1))
    start(gather_copies(1))
    wait(gather_copies(0))
    start(write_copies(0, 0))
    start(idx_copies(2, 0))

    def body(j, carry):
        i0 = 2 * j
        wait(idx_copies(i0, 0))
        wait(write_copies(i0 - 2, 0))
        start(gather_copies(0))
        wait(gather_copies(1))
        start(write_copies(i0 - 1, 1))
        start(idx_copies(i0 + 1, 1))
        wait(idx_copies(i0 + 1, 1))
        wait(write_copies(i0 - 1, 1))
        start(gather_copies(1))
        wait(gather_copies(0))
        start(write_copies(i0, 0))

        @pl.when(j < SHALF - 1)
        def _():
            start(idx_copies(i0 + 2, 0))

        return carry

    lax.fori_loop(1, SHALF, body, 0)

    wait(gather_copies(1))
    start(write_copies(STEPS - 1, 1))
    wait(write_copies(STEPS - 2, 0))
    wait(write_copies(STEPS - 1, 1))


@functools.partial(
    pl.kernel,
    out_type=jax.ShapeDtypeStruct((B, L, OUT_DIM), jnp.float32),
    mesh=plsc.VectorSubcoreMesh(core_axis_name="c", subcore_axis_name="s"),
    compiler_params=pltpu.CompilerParams(use_tc_tiling_on_sc=False),
    scratch_types=[
        pltpu.VMEM((2, L), jnp.int32),
        pltpu.VMEM((2, L), jnp.int32),
        pltpu.VMEM((2, L), jnp.int32),
        pltpu.VMEM((2, L, EMBED_DIM), jnp.float32),
        pltpu.VMEM((2, L, POS_DIM), jnp.float32),
        pltpu.VMEM((2, L, POS_DIM), jnp.float32),
        pltpu.SemaphoreType.DMA((2,)),
        pltpu.SemaphoreType.DMA((2,)),
        pltpu.SemaphoreType.DMA((2,)),
    ],
)
def _emb_kernel(*refs):
    _emb_body(*refs)


def kernel(word_id, pos_1, pos_2, word_table, pos_table):
    return _emb_kernel(word_id, pos_1, pos_2, word_table, pos_table)
